# baseline (speedup 1.0000x reference)
import functools

import jax
import jax.numpy as jnp
from jax import lax
from jax.experimental import pallas as pl
from jax.experimental.pallas import tpu as pltpu
from jax.experimental.pallas import tpu_sc as plsc

N = 16384
D = 2048
E = 16
TILE = 2048
GRID = N // TILE
EPS = 1e-6

NC = 2
NS = 16
NW = NC * NS
L = 16
TPW = N // NW
NG = TPW // L
UNROLL = 4
NOUT = NG // UNROLL


def _logits_kernel(x_ref, w_ref, out_ref):
    out_ref[...] = lax.dot_general(
        w_ref[...], x_ref[...], (((1,), (1,)), ((), ())),
        preferred_element_type=jnp.float32,
    )


def _sc_router(lg_hbm, gates_hbm, imp_hbm, load_hbm, lbuf, gbuf, ibuf, lvec,
               idxbuf):
    wid = lax.axis_index("s") * NC + lax.axis_index("c")
    base_tok = wid * TPW
    pltpu.sync_copy(lg_hbm.at[:, pl.ds(base_tok, TPW)], lbuf)

    iota = lax.iota(jnp.int32, L)

    def outer(u, carry):
        for gg in range(UNROLL):
            base = u * (UNROLL * L) + gg * L
            l = [lbuf[e, pl.ds(base, L)] for e in range(E)]
            m = l[0]
            idx = jnp.zeros((L,), jnp.int32)
            for e in range(1, E):
                gt = l[e] > m
                idx = jnp.where(gt, jnp.full((L,), e, jnp.int32), idx)
                m = jnp.maximum(m, l[e])
            s = jnp.zeros((L,), jnp.float32)
            pe = []
            for e in range(E):
                ee = jnp.exp(l[e] - m)
                pe.append(ee)
                s = s + ee
            r = 1.0 / s
            ld = carry[0]
            accs = list(carry[1])
            for e in range(E):
                accs[e] = accs[e] + pe[e] * r
            idxbuf[:] = idx
            for k in range(L):
                kb = jnp.full((L,), k, jnp.int32)
                idk = plsc.load_gather(idxbuf, [kb])
                row = (iota == idk).astype(jnp.float32)
                gbuf[pl.ds((base + k) * E, L)] = row
                ld = ld + row
            carry = (ld, tuple(accs))
        return carry

    zf = jnp.zeros((L,), jnp.float32)
    ld, accs = lax.fori_loop(
        0, NOUT, outer, (zf, tuple(zf for _ in range(E)))
    )
    for e in range(E):
        ibuf[e, :] = accs[e]
    lvec[:] = ld
    pltpu.sync_copy(gbuf, gates_hbm.at[pl.ds(base_tok * E, TPW * E)])
    pltpu.sync_copy(ibuf, imp_hbm.at[wid])
    pltpu.sync_copy(lvec, load_hbm.at[wid])


_sc_router_call = functools.partial(
    pl.kernel,
    out_type=[
        jax.ShapeDtypeStruct((N * E,), jnp.float32),
        jax.ShapeDtypeStruct((NW, E, L), jnp.float32),
        jax.ShapeDtypeStruct((NW, L), jnp.float32),
    ],
    mesh=plsc.VectorSubcoreMesh(core_axis_name="c", subcore_axis_name="s"),
    compiler_params=pltpu.CompilerParams(needs_layout_passes=False),
    scratch_types=[
        pltpu.VMEM((E, TPW), jnp.float32),
        pltpu.VMEM((TPW * E,), jnp.float32),
        pltpu.VMEM((E, L), jnp.float32),
        pltpu.VMEM((L,), jnp.float32),
        pltpu.VMEM((L,), jnp.int32),
    ],
)(_sc_router)


def _aux_kernel(ua_ref, imp_ref, load_ref, aux_ref):
    ip = jnp.sum(imp_ref[...], axis=(0, 2))
    ld = jnp.sum(load_ref[...], axis=0)
    ipn = ip / (jnp.sum(ip) + EPS)
    ldn = ld / (jnp.sum(ld) + EPS)
    d2 = (ipn - ldn) ** 2
    aux_ref[...] = (jnp.sum(d2) / E * ua_ref[0, 0]).reshape(1, 1)


def kernel(x, W, use_aux_loss):
    ua = jnp.asarray(use_aux_loss, jnp.float32).reshape(1, 1)
    lg = pl.pallas_call(
        _logits_kernel,
        grid=(GRID,),
        in_specs=[
            pl.BlockSpec((TILE, D), lambda i: (i, 0)),
            pl.BlockSpec((E, D), lambda i: (0, 0)),
        ],
        out_specs=pl.BlockSpec((E, TILE), lambda i: (0, i)),
        out_shape=jax.ShapeDtypeStruct((E, N), jnp.float32),
        compiler_params=pltpu.CompilerParams(
            dimension_semantics=("arbitrary",)
        ),
    )(x, W)
    gates_flat, imp3, load2 = _sc_router_call(lg)
    aux = pl.pallas_call(
        _aux_kernel,
        in_specs=[
            pl.BlockSpec(memory_space=pltpu.SMEM),
            pl.BlockSpec(memory_space=pltpu.VMEM),
            pl.BlockSpec(memory_space=pltpu.VMEM),
        ],
        out_specs=pl.BlockSpec(memory_space=pltpu.VMEM),
        out_shape=jax.ShapeDtypeStruct((1, 1), jnp.float32),
    )(ua, imp3, load2)
    return gates_flat.reshape(N, E), aux.reshape(())


# speedup vs baseline: 1.4815x; 1.4815x over previous
import jax
import jax.numpy as jnp
from jax import lax
from jax.experimental import pallas as pl
from jax.experimental.pallas import tpu as pltpu

N = 16384
D = 2048
E = 16
TILE = 1024
GRID = N // TILE
EPS = 1e-6


def _router_kernel(ua_ref, x_ref, w_ref, gates_ref, aux_ref, imp_ref, load_ref):
    i = pl.program_id(0)
    lt = lax.dot_general(
        w_ref[...], x_ref[...], (((1,), (1,)), ((), ())),
        preferred_element_type=jnp.float32,
    )
    logits = lt.T
    m = jnp.max(logits, axis=1, keepdims=True)
    e = jnp.exp(logits - m)
    s = jnp.sum(e, axis=1, keepdims=True)
    probs = e / s
    ids = lax.broadcasted_iota(jnp.int32, (TILE, E), 1)
    ismax = logits == m
    first = jnp.min(jnp.where(ismax, ids, E), axis=1, keepdims=True)
    gates = (ids == first).astype(jnp.float32)
    gates_ref[...] = gates
    imp_part = jnp.sum(probs, axis=0, keepdims=True)
    load_part = jnp.sum(gates, axis=0, keepdims=True)

    @pl.when(i == 0)
    def _():
        imp_ref[...] = imp_part
        load_ref[...] = load_part

    @pl.when(i > 0)
    def _():
        imp_ref[...] += imp_part
        load_ref[...] += load_part

    @pl.when(i == GRID - 1)
    def _():
        imp = imp_ref[...]
        ld = load_ref[...]
        impn = imp / (jnp.sum(imp) + EPS)
        ldn = ld / (jnp.sum(ld) + EPS)
        d2 = (impn - ldn) ** 2
        aux_ref[...] = jnp.sum(d2, axis=1, keepdims=True) / E * ua_ref[0, 0]


def kernel(x, W, use_aux_loss):
    ua = jnp.asarray(use_aux_loss, jnp.float32).reshape(1, 1)
    gates, aux = pl.pallas_call(
        _router_kernel,
        grid=(GRID,),
        in_specs=[
            pl.BlockSpec(memory_space=pltpu.SMEM),
            pl.BlockSpec((TILE, D), lambda i: (i, 0)),
            pl.BlockSpec((E, D), lambda i: (0, 0)),
        ],
        out_specs=[
            pl.BlockSpec((TILE, E), lambda i: (i, 0)),
            pl.BlockSpec((1, 1), lambda i: (0, 0)),
        ],
        out_shape=[
            jax.ShapeDtypeStruct((N, E), jnp.float32),
            jax.ShapeDtypeStruct((1, 1), jnp.float32),
        ],
        scratch_shapes=[
            pltpu.VMEM((1, E), jnp.float32),
            pltpu.VMEM((1, E), jnp.float32),
        ],
        compiler_params=pltpu.CompilerParams(
            dimension_semantics=("arbitrary",)
        ),
    )(ua, x, W)
    return gates, aux.reshape(())


# speedup vs baseline: 1.5231x; 1.0281x over previous
import jax
import jax.numpy as jnp
from jax import lax
from jax.experimental import pallas as pl
from jax.experimental.pallas import tpu as pltpu

N = 16384
D = 2048
E = 16
TILE = 2048
GRID = N // TILE
EPS = 1e-6


def _router_kernel(ua_ref, x_ref, w_ref, gates_ref, aux_ref, imp_ref, load_ref):
    i = pl.program_id(0)
    lt = lax.dot_general(
        w_ref[...], x_ref[...], (((1,), (1,)), ((), ())),
        preferred_element_type=jnp.float32,
    )
    logits = lt.T
    m = jnp.max(logits, axis=1, keepdims=True)
    e = jnp.exp(logits - m)
    s = jnp.sum(e, axis=1, keepdims=True)
    probs = e / s
    ids = lax.broadcasted_iota(jnp.int32, (TILE, E), 1)
    ismax = logits == m
    first = jnp.min(jnp.where(ismax, ids, E), axis=1, keepdims=True)
    gates = (ids == first).astype(jnp.float32)
    gates_ref[...] = gates
    imp_part = jnp.sum(probs, axis=0, keepdims=True)
    load_part = jnp.sum(gates, axis=0, keepdims=True)

    @pl.when(i == 0)
    def _():
        imp_ref[...] = imp_part
        load_ref[...] = load_part

    @pl.when(i > 0)
    def _():
        imp_ref[...] += imp_part
        load_ref[...] += load_part

    @pl.when(i == GRID - 1)
    def _():
        imp = imp_ref[...]
        ld = load_ref[...]
        impn = imp / (jnp.sum(imp) + EPS)
        ldn = ld / (jnp.sum(ld) + EPS)
        d2 = (impn - ldn) ** 2
        aux_ref[...] = jnp.sum(d2, axis=1, keepdims=True) / E * ua_ref[0, 0]


def kernel(x, W, use_aux_loss):
    ua = jnp.asarray(use_aux_loss, jnp.float32).reshape(1, 1)
    gates, aux = pl.pallas_call(
        _router_kernel,
        grid=(GRID,),
        in_specs=[
            pl.BlockSpec(memory_space=pltpu.SMEM),
            pl.BlockSpec((TILE, D), lambda i: (i, 0)),
            pl.BlockSpec((E, D), lambda i: (0, 0)),
        ],
        out_specs=[
            pl.BlockSpec((TILE, E), lambda i: (i, 0)),
            pl.BlockSpec((1, 1), lambda i: (0, 0)),
        ],
        out_shape=[
            jax.ShapeDtypeStruct((N, E), jnp.float32),
            jax.ShapeDtypeStruct((1, 1), jnp.float32),
        ],
        scratch_shapes=[
            pltpu.VMEM((1, E), jnp.float32),
            pltpu.VMEM((1, E), jnp.float32),
        ],
        compiler_params=pltpu.CompilerParams(
            dimension_semantics=("arbitrary",)
        ),
    )(ua, x, W)
    return gates, aux.reshape(())
